# baseline (speedup 1.0000x reference)
import functools

import numpy as np
import jax
import jax.numpy as jnp
from jax import lax
from jax.experimental import pallas as pl
from jax.experimental.pallas import tpu as pltpu
from jax.experimental.pallas import tpu_sc as plsc

SC_NC = 2
SC_NS = 16
SC_NW = SC_NC * SC_NS

HID = 64
BN_EPS = 1e-5
SEG_EPS = 1e-6


def _row_block(n, cap=2048):
    r = 8
    for c in range(8, cap + 1, 8):
        if n % c == 0:
            r = c
    return r


def _grid_call(kern, n, ins, in_widths, out_widths, n_stats, row_block=None):
    R = row_block or _row_block(n)
    grid = n // R
    in_specs = []
    for a, w in zip(ins, in_widths):
        if w is None:
            in_specs.append(
                pl.BlockSpec(a.shape, lambda i, nd=a.ndim: (0,) * nd))
        else:
            in_specs.append(pl.BlockSpec((R, w), lambda i: (i, 0)))
    out_specs = [pl.BlockSpec((R, w), lambda i: (i, 0)) for w in out_widths]
    out_shape = [jax.ShapeDtypeStruct((n, w), jnp.float32) for w in out_widths]
    for _ in range(n_stats):
        out_specs.append(pl.BlockSpec((2, HID), lambda i: (0, 0)))
        out_shape.append(jax.ShapeDtypeStruct((2, HID), jnp.float32))
    outs = pl.pallas_call(
        functools.partial(kern, grid=grid),
        grid=(grid,),
        in_specs=in_specs,
        out_specs=out_specs,
        out_shape=out_shape,
        scratch_shapes=[pltpu.VMEM((2, HID), jnp.float32)] * n_stats,
    )(*ins)
    return outs


def _accum_stats(t, i, grid, s_ref, acc_ref):
    ps = jnp.concatenate(
        [jnp.sum(t, axis=0, keepdims=True),
         jnp.sum(t * t, axis=0, keepdims=True)], axis=0)

    @pl.when(i == 0)
    def _():
        acc_ref[...] = ps

    @pl.when(i > 0)
    def _():
        acc_ref[...] = acc_ref[...] + ps

    @pl.when(i == grid - 1)
    def _():
        s_ref[...] = acc_ref[...]


def _bn_apply(t, s, n):
    if s.ndim == 3:
        s = jnp.sum(s, axis=0)
    mu = s[0:1, :] / n
    var = s[1:2, :] / n - mu * mu
    return (t - mu) * jax.lax.rsqrt(var + BN_EPS)


def _silu(v):
    return v * jax.nn.sigmoid(v)


def _mm_stats(x, W, b):
    n = x.shape[0]

    def kern(x_ref, w_ref, b_ref, t_ref, s_ref, acc_ref, *, grid):
        i = pl.program_id(0)
        t = jnp.dot(x_ref[...], w_ref[...],
                    preferred_element_type=jnp.float32) + b_ref[...]
        t_ref[...] = t
        _accum_stats(t, i, grid, s_ref, acc_ref)

    t, s = _grid_call(kern, n, [x, W, b.reshape(1, -1)],
                      [x.shape[1], None, None], [HID], 1)
    return t, s


def _rbf_mm_stats(d2col, W, b, vmin, vmax, bins, is_r):
    n = d2col.shape[0]
    centers = jnp.asarray(
        np.linspace(vmin, vmax, bins, dtype=np.float32)).reshape(1, bins)
    gamma = 1.0 / float(np.diff(np.linspace(vmin, vmax, bins)).mean())

    def kern(d_ref, c_ref, w_ref, b_ref, t_ref, s_ref, acc_ref, *, grid):
        i = pl.program_id(0)
        db = d_ref[...]
        if is_r:
            db = jnp.sqrt(jnp.sum(db * db, axis=1, keepdims=True))
        rbf = jnp.exp(-gamma * (db - c_ref[...]) ** 2)
        t = jnp.dot(rbf, w_ref[...],
                    preferred_element_type=jnp.float32) + b_ref[...]
        t_ref[...] = t
        _accum_stats(t, i, grid, s_ref, acc_ref)

    t, s = _grid_call(kern, n, [d2col, centers, W, b.reshape(1, -1)],
                      [d2col.shape[1], None, None, None], [HID], 1)
    return t, s


def _bnsilu_mm_stats(t1, s1, W, b):
    n = t1.shape[0]

    def kern(t1_ref, s1_ref, w_ref, b_ref, t_ref, s_ref, acc_ref, *, grid):
        i = pl.program_id(0)
        u = _silu(_bn_apply(t1_ref[...], s1_ref[...], n))
        t = jnp.dot(u, w_ref[...],
                    preferred_element_type=jnp.float32) + b_ref[...]
        t_ref[...] = t
        _accum_stats(t, i, grid, s_ref, acc_ref)

    t, s = _grid_call(kern, n, [t1, s1, W, b.reshape(1, -1)],
                      [HID, None, None, None], [HID], 1)
    return t, s


def _bnsilu(t, s):
    n = t.shape[0]

    def kern(t_ref, s_ref, o_ref, *, grid):
        o_ref[...] = _silu(_bn_apply(t_ref[...], s_ref[...], n))

    (o,) = _grid_call(kern, n, [t, s], [HID, None], [HID], 0)
    return o


def _residual_bnsilu(t, s, res):
    n = t.shape[0]

    def kern(t_ref, s_ref, r_ref, o_ref, *, grid):
        o_ref[...] = r_ref[...] + _silu(_bn_apply(t_ref[...], s_ref[...], n))

    (o,) = _grid_call(kern, n, [t, s, res], [HID, None, HID], [HID], 0)
    return o


def _mm3(x, Wcat, bcat):
    n = x.shape[0]

    def kern(x_ref, w_ref, b_ref, o1_ref, o2_ref, *, grid):
        t = jnp.dot(x_ref[...], w_ref[...],
                    preferred_element_type=jnp.float32) + b_ref[...]
        o1_ref[...] = t[:, :2 * HID]
        o2_ref[...] = t[:, 2 * HID:]

    o1, o2 = _grid_call(kern, n, [x, Wcat, bcat.reshape(1, -1)],
                        [HID, None, None], [2 * HID, 2 * HID], 0)
    return o1, o2


def _mm(x, W, b):
    n = x.shape[0]

    def kern(x_ref, w_ref, b_ref, o_ref, *, grid):
        o_ref[...] = jnp.dot(x_ref[...], w_ref[...],
                             preferred_element_type=jnp.float32) + b_ref[...]

    (o,) = _grid_call(kern, n, [x, W, b.reshape(1, -1)],
                      [HID, None, None], [HID], 0)
    return o


def _add_div_stats(T_dx, parts):
    n = T_dx.shape[0]

    def kern(x_ref, *refs, grid):
        part_refs = refs[:len(parts)]
        t_ref, s_ref, acc_ref = refs[len(parts):]
        i = pl.program_id(0)
        ss = part_refs[0][...]
        for pr in part_refs[1:]:
            ss = ss + pr[...]
        t = x_ref[:, HID:] + ss[:, HID:] / (ss[:, :HID] + SEG_EPS)
        t_ref[...] = t
        _accum_stats(t, i, grid, s_ref, acc_ref)

    t, s = _grid_call(kern, n, [T_dx] + list(parts),
                      [2 * HID] + [2 * HID] * len(parts), [HID], 1)
    return t, s


def _readout(x, W_fc, b_fc):
    n = x.shape[0]
    R = _row_block(n)
    grid = n // R

    def kern(x_ref, w_ref, b_ref, o_ref, acc_ref):
        i = pl.program_id(0)
        ps = jnp.sum(x_ref[...], axis=0, keepdims=True)

        @pl.when(i == 0)
        def _():
            acc_ref[...] = ps

        @pl.when(i > 0)
        def _():
            acc_ref[...] = acc_ref[...] + ps

        @pl.when(i == grid - 1)
        def _():
            h = acc_ref[...] / n
            o_ref[...] = jnp.dot(h, w_ref[...],
                                 preferred_element_type=jnp.float32) + b_ref[...]

    out = pl.pallas_call(
        kern,
        grid=(grid,),
        in_specs=[pl.BlockSpec((R, HID), lambda i: (i, 0)),
                  pl.BlockSpec((HID, 1), lambda i: (0, 0)),
                  pl.BlockSpec((1, 1), lambda i: (0, 0))],
        out_specs=pl.BlockSpec((1, 1), lambda i: (0, 0)),
        out_shape=jax.ShapeDtypeStruct((1, 1), jnp.float32),
        scratch_shapes=[pltpu.VMEM((1, HID), jnp.float32)],
    )(x, W_fc, b_fc.reshape(1, 1))
    return jnp.squeeze(out)


def _edge_fused(T_sg, T_dx, yW2, src, dst):
    E = src.shape[0]
    C = 128
    C2 = C // 2
    assert E % (2 * C) == 0
    nchunks = E // C
    sd = jnp.stack([src.reshape(nchunks, C), dst.reshape(nchunks, C)], 1)
    y2 = yW2.reshape(E // 2, 2 * HID)
    w_nj = nchunks // SC_NW
    w_extra = nchunks - w_nj * SC_NW
    mesh = plsc.VectorSubcoreMesh(core_axis_name="c", subcore_axis_name="s")

    @functools.partial(
        pl.kernel, mesh=mesh,
        out_type=[jax.ShapeDtypeStruct((E // 2, 2 * HID), jnp.float32),
                  jax.ShapeDtypeStruct((SC_NW, 2 * HID), jnp.float32),
                  jax.ShapeDtypeStruct((E, 2 * HID), jnp.float32)],
        scratch_types=[pltpu.VMEM((2, 2, C), jnp.int32),
                       pltpu.VMEM((2 * C, 2 * HID), jnp.float32),
                       pltpu.VMEM((2 * C, 2 * HID), jnp.float32),
                       pltpu.VMEM((C, 2 * HID), jnp.float32),
                       pltpu.VMEM((2 * HID,), jnp.float32),
                       pltpu.SemaphoreType.DMA,
                       pltpu.SemaphoreType.DMA,
                       pltpu.SemaphoreType.DMA,
                       pltpu.SemaphoreType.DMA],
    )
    def k(tsg_hbm, tdx_hbm, y_hbm, sd_hbm, m_hbm, st_hbm, sp_hbm,
          idx, rows1, rows2, ybuf, statb, sem0, sem1, sem_w, sem_s):
        c = lax.axis_index("c")
        s = lax.axis_index("s")
        wid = s * SC_NC + c
        for j in range(2 * HID // 16):
            statb[pl.ds(16 * j, 16)] = jnp.zeros((16,), jnp.float32)
        start = wid * w_nj + jnp.minimum(wid, w_extra)
        nj = w_nj + jnp.where(wid < w_extra, 1, 0)

        def compute_chunk(slot):
            base = slot * C
            ybase = slot * C2

            def body(i, carry):
                accs = list(carry)
                for half in range(2):
                    r = base + 2 * i + half
                    for j in range(4):
                        g1 = rows1[r, pl.ds(16 * j, 16)]
                        g2 = rows2[r, pl.ds(16 * j, 16)]
                        yv = ybuf[ybase + i, pl.ds(64 * half + 16 * j, 16)]
                        mv = g1 + g2 + yv
                        sig = 1.0 / (1.0 + jnp.exp(-mv))
                        bh = rows1[r, pl.ds(64 + 16 * j, 16)]
                        ybuf[ybase + i, pl.ds(64 * half + 16 * j, 16)] = mv
                        rows1[r, pl.ds(16 * j, 16)] = sig
                        rows1[r, pl.ds(64 + 16 * j, 16)] = bh * sig
                        accs[j] = accs[j] + mv
                        accs[4 + j] = accs[4 + j] + mv * mv
                return tuple(accs)

            fin = pl.loop(0, C2, init_carry=tuple(
                jnp.zeros((16,), jnp.float32) for _ in range(8)))(body)
            for j in range(8):
                statb[pl.ds(16 * j, 16)] = statb[pl.ds(16 * j, 16)] + fin[j]

        def emit_out(slot, kk):
            wm = pltpu.async_copy(ybuf.at[pl.ds(slot * C2, C2)],
                                  m_hbm.at[pl.ds(kk * C2, C2)], sem_w)
            ws = pltpu.async_copy(rows1.at[pl.ds(slot * C, C)],
                                  sp_hbm.at[pl.ds(kk * C, C)], sem_s)
            return wm, ws

        @pl.loop(0, nj // 2)
        def _pair(j2):
            ka = start + 2 * j2
            kb = ka + 1
            ia = pltpu.async_copy(sd_hbm.at[ka], idx.at[0], sem0)
            ya = pltpu.async_copy(y_hbm.at[pl.ds(ka * C2, C2)],
                                  ybuf.at[pl.ds(0, C2)], sem0)
            ib = pltpu.async_copy(sd_hbm.at[kb], idx.at[1], sem1)
            yb = pltpu.async_copy(y_hbm.at[pl.ds(kb * C2, C2)],
                                  ybuf.at[pl.ds(C2, C2)], sem1)
            ia.wait()
            ya.wait()
            g1a = pltpu.async_copy(tsg_hbm.at[idx.at[0, 0]],
                                   rows1.at[pl.ds(0, C)], sem0)
            g2a = pltpu.async_copy(tdx_hbm.at[idx.at[0, 1]],
                                   rows2.at[pl.ds(0, C)], sem0)
            ib.wait()
            yb.wait()
            g1b = pltpu.async_copy(tsg_hbm.at[idx.at[1, 0]],
                                   rows1.at[pl.ds(C, C)], sem1)
            g2b = pltpu.async_copy(tdx_hbm.at[idx.at[1, 1]],
                                   rows2.at[pl.ds(C, C)], sem1)
            g1a.wait()
            g2a.wait()
            compute_chunk(0)
            wma, wsa = emit_out(0, ka)
            g1b.wait()
            g2b.wait()
            compute_chunk(1)
            wmb, wsb = emit_out(1, kb)
            wma.wait()
            wsa.wait()
            wmb.wait()
            wsb.wait()

        @pl.when(nj % 2 == 1)
        def _tail():
            kk = start + nj - 1
            pltpu.sync_copy(sd_hbm.at[kk], idx.at[0])
            pltpu.sync_copy(y_hbm.at[pl.ds(kk * C2, C2)],
                            ybuf.at[pl.ds(0, C2)])
            ga = pltpu.async_copy(tsg_hbm.at[idx.at[0, 0]],
                                  rows1.at[pl.ds(0, C)], sem0)
            gb = pltpu.async_copy(tdx_hbm.at[idx.at[0, 1]],
                                  rows2.at[pl.ds(0, C)], sem1)
            ga.wait()
            gb.wait()
            compute_chunk(0)
            wm, ws = emit_out(0, kk)
            wm.wait()
            ws.wait()

        pltpu.sync_copy(statb, st_hbm.at[wid])

    return k(T_sg, T_dx, y2, sd)


_LG_BS = 4096
_LG_TRASH = 128


def _sc_scatter_lg(SP, sorted_eid, sorted_dst, starts, n):
    E = SP.shape[0]
    C = 128
    BS = _LG_BS
    assert E % C == 0
    nchunks = E // C
    nbins = -(-n // BS)
    nbins = nbins + nbins % SC_NC
    passes = nbins // SC_NC
    stripe = (BS + _LG_TRASH) // SC_NS
    dump = BS // SC_NS
    zeros = jnp.zeros((stripe, 2 * HID), jnp.float32)
    eid2 = sorted_eid.reshape(nchunks, C)
    sdst2 = sorted_dst.reshape(nchunks, C)
    mesh = plsc.VectorSubcoreMesh(core_axis_name="c", subcore_axis_name="s")

    @functools.partial(
        pl.kernel, mesh=mesh,
        out_type=jax.ShapeDtypeStruct((nbins * BS, 2 * HID), jnp.float32),
        scratch_types=[pltpu.VMEM((2, C), jnp.int32),
                       pltpu.VMEM((2, C), jnp.int32),
                       pltpu.VMEM((2, C), jnp.int32),
                       pltpu.VMEM((2 * C, 2 * HID), jnp.float32),
                       pltpu.VMEM_SHARED((BS + _LG_TRASH, 2 * HID),
                                         jnp.float32),
                       pltpu.VMEM((nbins, 16), jnp.int32),
                       pltpu.SemaphoreType.DMA,
                       pltpu.SemaphoreType.DMA,
                       pltpu.SemaphoreType.DMA],
    )
    def k(sp_hbm, eid_hbm, sdst_hbm, starts_hbm, z_hbm, out_hbm,
          idxd, idxe, locb, rows, acc, st_v, sem0, sem1, sem_s):
        c = lax.axis_index("c")
        s = lax.axis_index("s")
        pltpu.sync_copy(starts_hbm, st_v)

        def compute_loc(buf, base_val):
            for i in range(C // 16):
                v = idxd[buf, pl.ds(i * 16, 16)]
                lv = v - base_val
                valid = (lv >= 0) & (lv < BS)
                trash = BS + i * 16 + lax.iota(jnp.int32, 16)
                locb[buf, pl.ds(i * 16, 16)] = jnp.where(valid, lv, trash)

        @pl.loop(0, passes)
        def _pass(p):
            b = p * SC_NC + c
            base_val = b * BS
            pltpu.sync_copy(z_hbm, acc.at[pl.ds(s * stripe, stripe)])
            plsc.subcore_barrier()
            row = st_v[b]
            lo = row[0]
            hi = row[1]
            c0 = lo // C
            c1 = (hi + C - 1) // C
            nj = jnp.maximum(0, (c1 - c0 - s + SC_NS - 1) // SC_NS)

            @pl.loop(0, nj // 2)
            def _pair(j2):
                ka = c0 + s + SC_NS * 2 * j2
                kb = ka + SC_NS
                da = pltpu.async_copy(sdst_hbm.at[ka], idxd.at[0], sem0)
                ea = pltpu.async_copy(eid_hbm.at[ka], idxe.at[0], sem0)
                db = pltpu.async_copy(sdst_hbm.at[kb], idxd.at[1], sem1)
                eb = pltpu.async_copy(eid_hbm.at[kb], idxe.at[1], sem1)
                da.wait()
                ea.wait()
                ga = pltpu.async_copy(sp_hbm.at[idxe.at[0]],
                                      rows.at[pl.ds(0, C)], sem0)
                db.wait()
                eb.wait()
                gb = pltpu.async_copy(sp_hbm.at[idxe.at[1]],
                                      rows.at[pl.ds(C, C)], sem1)
                compute_loc(0, base_val)
                ga.wait()
                sa = pltpu.async_copy(rows.at[pl.ds(0, C)],
                                      acc.at[locb.at[0]], sem_s, add=True)
                compute_loc(1, base_val)
                gb.wait()
                sb = pltpu.async_copy(rows.at[pl.ds(C, C)],
                                      acc.at[locb.at[1]], sem_s, add=True)
                sa.wait()
                sb.wait()

            @pl.when(nj % 2 == 1)
            def _tail():
                kk = c0 + s + SC_NS * (nj - 1)
                pltpu.sync_copy(sdst_hbm.at[kk], idxd.at[0])
                pltpu.sync_copy(eid_hbm.at[kk], idxe.at[0])
                cp = pltpu.async_copy(sp_hbm.at[idxe.at[0]],
                                      rows.at[pl.ds(0, C)], sem0)
                compute_loc(0, base_val)
                cp.wait()
                pltpu.sync_copy(rows.at[pl.ds(0, C)], acc.at[locb.at[0]],
                                add=True)

            plsc.subcore_barrier()
            pltpu.sync_copy(
                acc.at[pl.ds(s * dump, dump)],
                out_hbm.at[pl.ds(pl.multiple_of(base_val + s * dump, dump),
                                 dump)])
            plsc.subcore_barrier()

    return k(SP, eid2, sdst2, starts, zeros)


def _egc_layer(x, y, W, b, src, dst, n, lg_sort):
    Wcat = jnp.concatenate([W[0], W[4], W[1], W[3]], axis=1)
    bcat = jnp.concatenate([b[0], b[4], b[1], b[3]])
    T_sg, T_dx = _mm3(x, Wcat, bcat)
    yW2 = _mm(y, W[2], b[2])
    m2, stats_m, SP = _edge_fused(T_sg, T_dx, yW2, src, dst)
    sorted_eid, sorted_dst, starts = lg_sort
    parts = [_sc_scatter_lg(SP, sorted_eid, sorted_dst, starts, n)]
    m = m2.reshape(-1, HID)
    stats_m = stats_m.reshape(SC_NW, 2, HID)
    t, stats_t = _add_div_stats(T_dx, parts)
    x_new = _residual_bnsilu(t, stats_t, x)
    y_new = _residual_bnsilu(m, stats_m, y)
    return x_new, y_new


def kernel(atom_features, r, angle_h, edge_index, lg_edge_index, W_atom,
           b_atom, W_e1, b_e1, W_e2, b_e2, W_a1, b_a1, W_a2, b_a2, egc_W,
           egc_b, W_fc, b_fc):
    src, dst = edge_index[0], edge_index[1]
    lsrc, ldst = lg_edge_index[0], lg_edge_index[1]
    N = atom_features.shape[0]
    E = r.shape[0]

    t, s = _rbf_mm_stats(angle_h.reshape(-1, 1), W_a1, b_a1, -1.0, 1.0, 40,
                         is_r=False)
    t, s = _bnsilu_mm_stats(t, s, W_a2, b_a2)
    z = _bnsilu(t, s)

    t, s = _mm_stats(atom_features, W_atom, b_atom)
    x = _bnsilu(t, s)

    t, s = _rbf_mm_stats(r, W_e1, b_e1, 0.0, 8.0, 80, is_r=True)
    t, s = _bnsilu_mm_stats(t, s, W_e2, b_e2)
    y = _bnsilu(t, s)

    def _sort_meta(d, nseg):
        ne = d.shape[0]
        nbins = -(-nseg // _LG_BS)
        nbins = nbins + nbins % SC_NC
        sd, perm = lax.sort_key_val(d, jnp.arange(ne, dtype=jnp.int32))
        st = jnp.searchsorted(
            sd, jnp.arange(nbins + 1, dtype=jnp.int32) * _LG_BS
        ).astype(jnp.int32)
        st2 = jnp.zeros((nbins, 16), jnp.int32)
        st2 = st2.at[:, 0].set(st[:-1]).at[:, 1].set(st[1:])
        return (perm, sd, st2)

    g_sort = _sort_meta(dst, N)
    lg_sort = _sort_meta(ldst, E)

    x, m = _egc_layer(x, y, egc_W[0], egc_b[0], src, dst, N, g_sort)
    y, z = _egc_layer(m, z, egc_W[1], egc_b[1], lsrc, ldst, E, lg_sort)
    x, m = _egc_layer(x, y, egc_W[2], egc_b[2], src, dst, N, g_sort)
    y, z = _egc_layer(m, z, egc_W[3], egc_b[3], lsrc, ldst, E, lg_sort)
    x, y = _egc_layer(x, y, egc_W[4], egc_b[4], src, dst, N, g_sort)
    x, y = _egc_layer(x, y, egc_W[5], egc_b[5], src, dst, N, g_sort)

    return _readout(x, W_fc, b_fc)


# speedup vs baseline: 1.3786x; 1.3786x over previous
import functools

import numpy as np
import jax
import jax.numpy as jnp
from jax import lax
from jax.experimental import pallas as pl
from jax.experimental.pallas import tpu as pltpu
from jax.experimental.pallas import tpu_sc as plsc

SC_NC = 2
SC_NS = 16
SC_NW = SC_NC * SC_NS

HID = 64
BN_EPS = 1e-5
SEG_EPS = 1e-6


def _row_block(n, cap=8000):
    r = 8
    for c in range(8, cap + 1, 8):
        if n % c == 0:
            r = c
    return r


def _grid_call(kern, n, ins, in_widths, out_widths, n_stats, row_block=None):
    R = row_block or _row_block(n)
    grid = n // R
    in_specs = []
    for a, w in zip(ins, in_widths):
        if w is None:
            in_specs.append(
                pl.BlockSpec(a.shape, lambda i, nd=a.ndim: (0,) * nd))
        else:
            in_specs.append(pl.BlockSpec((R, w), lambda i: (i, 0)))
    out_specs = [pl.BlockSpec((R, w), lambda i: (i, 0)) for w in out_widths]
    out_shape = [jax.ShapeDtypeStruct((n, w), jnp.float32) for w in out_widths]
    for _ in range(n_stats):
        out_specs.append(pl.BlockSpec((3, HID), lambda i: (0, 0)))
        out_shape.append(jax.ShapeDtypeStruct((3, HID), jnp.float32))
    outs = pl.pallas_call(
        functools.partial(kern, grid=grid),
        grid=(grid,),
        in_specs=in_specs,
        out_specs=out_specs,
        out_shape=out_shape,
        scratch_shapes=[pltpu.VMEM((3, HID), jnp.float32)] * n_stats,
    )(*ins)
    return outs


def _accum_stats(t, i, grid, s_ref, acc_ref):
    @pl.when(i == 0)
    def _():
        acc_ref[2:3, :] = t[0:1, :]

    td = t - acc_ref[2:3, :]
    ps = jnp.concatenate(
        [jnp.sum(td, axis=0, keepdims=True),
         jnp.sum(td * td, axis=0, keepdims=True)], axis=0)

    @pl.when(i == 0)
    def _():
        acc_ref[0:2, :] = ps

    @pl.when(i > 0)
    def _():
        acc_ref[0:2, :] = acc_ref[0:2, :] + ps

    @pl.when(i == grid - 1)
    def _():
        s_ref[...] = acc_ref[...]


def _bn_apply(t, s, n):
    mu_d = s[0:1, :] / n
    var = s[1:2, :] / n - mu_d * mu_d
    mu = s[2:3, :] + mu_d
    return (t - mu) * jax.lax.rsqrt(var + BN_EPS)


def _silu(v):
    return v * jax.nn.sigmoid(v)


def _mm_stats(x, W, b):
    n = x.shape[0]

    def kern(x_ref, w_ref, b_ref, t_ref, s_ref, acc_ref, *, grid):
        i = pl.program_id(0)
        t = jnp.dot(x_ref[...], w_ref[...],
                    preferred_element_type=jnp.float32,
                    precision=lax.Precision.HIGHEST) + b_ref[...]
        t_ref[...] = t
        _accum_stats(t, i, grid, s_ref, acc_ref)

    t, s = _grid_call(kern, n, [x, W, b.reshape(1, -1)],
                      [x.shape[1], None, None], [HID], 1)
    return t, s


def _rbf_mm_stats(d2col, W, b, vmin, vmax, bins, is_r):
    n = d2col.shape[0]
    centers = jnp.asarray(
        np.linspace(vmin, vmax, bins, dtype=np.float32)).reshape(1, bins)
    gamma = 1.0 / float(np.diff(np.linspace(vmin, vmax, bins)).mean())

    def kern(d_ref, c_ref, w_ref, b_ref, t_ref, s_ref, acc_ref, *, grid):
        i = pl.program_id(0)
        db = d_ref[...]
        if is_r:
            db = jnp.sqrt(jnp.sum(db * db, axis=1, keepdims=True))
        rbf = jnp.exp(-gamma * (db - c_ref[...]) ** 2)
        t = jnp.dot(rbf, w_ref[...],
                    preferred_element_type=jnp.float32,
                    precision=lax.Precision.HIGHEST) + b_ref[...]
        t_ref[...] = t
        _accum_stats(t, i, grid, s_ref, acc_ref)

    t, s = _grid_call(kern, n, [d2col, centers, W, b.reshape(1, -1)],
                      [d2col.shape[1], None, None, None], [HID], 1)
    return t, s


def _bnsilu_mm_stats(t1, s1, W, b):
    n = t1.shape[0]

    def kern(t1_ref, s1_ref, w_ref, b_ref, t_ref, s_ref, acc_ref, *, grid):
        i = pl.program_id(0)
        u = _silu(_bn_apply(t1_ref[...], s1_ref[...], n))
        t = jnp.dot(u, w_ref[...],
                    preferred_element_type=jnp.float32,
                    precision=lax.Precision.HIGHEST) + b_ref[...]
        t_ref[...] = t
        _accum_stats(t, i, grid, s_ref, acc_ref)

    t, s = _grid_call(kern, n, [t1, s1, W, b.reshape(1, -1)],
                      [HID, None, None, None], [HID], 1)
    return t, s


def _bnsilu(t, s):
    n = t.shape[0]

    def kern(t_ref, s_ref, o_ref, *, grid):
        o_ref[...] = _silu(_bn_apply(t_ref[...], s_ref[...], n))

    (o,) = _grid_call(kern, n, [t, s], [HID, None], [HID], 0)
    return o


def _residual_bnsilu(t, s, res):
    n = t.shape[0]

    def kern(t_ref, s_ref, r_ref, o_ref, *, grid):
        o_ref[...] = r_ref[...] + _silu(_bn_apply(t_ref[...], s_ref[...], n))

    (o,) = _grid_call(kern, n, [t, s, res], [HID, None, HID], [HID], 0)
    return o


def _mm3(x, Wcat, bcat):
    n = x.shape[0]

    def kern(x_ref, w_ref, b_ref, o1_ref, o2_ref, *, grid):
        t = jnp.dot(x_ref[...], w_ref[...],
                    preferred_element_type=jnp.float32,
                    precision=lax.Precision.HIGHEST) + b_ref[...]
        o1_ref[...] = t[:, :2 * HID]
        o2_ref[...] = t[:, 2 * HID:]

    o1, o2 = _grid_call(kern, n, [x, Wcat, bcat.reshape(1, -1)],
                        [HID, None, None], [2 * HID, 2 * HID], 0)
    return o1, o2


def _mm(x, W, b):
    n = x.shape[0]

    def kern(x_ref, w_ref, b_ref, o_ref, *, grid):
        o_ref[...] = jnp.dot(x_ref[...], w_ref[...],
                             preferred_element_type=jnp.float32,
                    precision=lax.Precision.HIGHEST) + b_ref[...]

    (o,) = _grid_call(kern, n, [x, W, b.reshape(1, -1)],
                      [HID, None, None], [HID], 0)
    return o


def _add_div_stats(T_dx, parts):
    n = T_dx.shape[0]

    def kern(x_ref, *refs, grid):
        part_refs = refs[:len(parts)]
        t_ref, s_ref, acc_ref = refs[len(parts):]
        i = pl.program_id(0)
        ss = part_refs[0][...]
        for pr in part_refs[1:]:
            ss = ss + pr[...]
        t = x_ref[:, HID:] + ss[:, HID:] / (ss[:, :HID] + SEG_EPS)
        t_ref[...] = t
        _accum_stats(t, i, grid, s_ref, acc_ref)

    t, s = _grid_call(kern, n, [T_dx] + list(parts),
                      [2 * HID] + [2 * HID] * len(parts), [HID], 1)
    return t, s


def _readout(x, W_fc, b_fc):
    n = x.shape[0]
    R = _row_block(n)
    grid = n // R

    def kern(x_ref, w_ref, b_ref, o_ref, acc_ref):
        i = pl.program_id(0)
        ps = jnp.sum(x_ref[...], axis=0, keepdims=True)

        @pl.when(i == 0)
        def _():
            acc_ref[...] = ps

        @pl.when(i > 0)
        def _():
            acc_ref[...] = acc_ref[...] + ps

        @pl.when(i == grid - 1)
        def _():
            h = acc_ref[...] / n
            o_ref[...] = jnp.dot(h, w_ref[...],
                                 preferred_element_type=jnp.float32,
                    precision=lax.Precision.HIGHEST) + b_ref[...]

    out = pl.pallas_call(
        kern,
        grid=(grid,),
        in_specs=[pl.BlockSpec((R, HID), lambda i: (i, 0)),
                  pl.BlockSpec((HID, 1), lambda i: (0, 0)),
                  pl.BlockSpec((1, 1), lambda i: (0, 0))],
        out_specs=pl.BlockSpec((1, 1), lambda i: (0, 0)),
        out_shape=jax.ShapeDtypeStruct((1, 1), jnp.float32),
        scratch_shapes=[pltpu.VMEM((1, HID), jnp.float32)],
    )(x, W_fc, b_fc.reshape(1, 1))
    return jnp.squeeze(out)


def _gather_rows(T_sg, T_dx, src, dst):
    E = src.shape[0]
    C = 128
    assert E % C == 0
    nchunks = E // C
    base_nj = nchunks // SC_NW
    extra = nchunks - base_nj * SC_NW
    sd = jnp.stack([src.reshape(nchunks, C), dst.reshape(nchunks, C)], 1)
    mesh = plsc.VectorSubcoreMesh(core_axis_name="c", subcore_axis_name="s")

    @functools.partial(
        pl.kernel, mesh=mesh,
        out_type=[jax.ShapeDtypeStruct((E, 2 * HID), jnp.float32),
                  jax.ShapeDtypeStruct((E, 2 * HID), jnp.float32)],
        scratch_types=[pltpu.VMEM((2, 2, C), jnp.int32),
                       pltpu.VMEM((2 * C, 2 * HID), jnp.float32),
                       pltpu.VMEM((2 * C, 2 * HID), jnp.float32),
                       pltpu.SemaphoreType.DMA,
                       pltpu.SemaphoreType.DMA,
                       pltpu.SemaphoreType.DMA],
    )
    def k(tsg_hbm, tdx_hbm, sd_hbm, g1_hbm, g2_hbm,
          idx, rows1, rows2, sem0, sem1, sem_w):
        wid = lax.axis_index("s") * SC_NC + lax.axis_index("c")
        start = wid * base_nj + jnp.minimum(wid, extra)
        nj = base_nj + jnp.where(wid < extra, 1, 0)

        @pl.loop(0, nj // 2)
        def _pair(j2):
            ka = start + 2 * j2
            kb = ka + 1
            ia = pltpu.async_copy(sd_hbm.at[ka], idx.at[0], sem0)
            ib = pltpu.async_copy(sd_hbm.at[kb], idx.at[1], sem1)
            ia.wait()
            g1a = pltpu.async_copy(tsg_hbm.at[idx.at[0, 0]],
                                   rows1.at[pl.ds(0, C)], sem0)
            g2a = pltpu.async_copy(tdx_hbm.at[idx.at[0, 1]],
                                   rows2.at[pl.ds(0, C)], sem0)
            ib.wait()
            g1b = pltpu.async_copy(tsg_hbm.at[idx.at[1, 0]],
                                   rows1.at[pl.ds(C, C)], sem1)
            g2b = pltpu.async_copy(tdx_hbm.at[idx.at[1, 1]],
                                   rows2.at[pl.ds(C, C)], sem1)
            g1a.wait()
            g2a.wait()
            w1a = pltpu.async_copy(rows1.at[pl.ds(0, C)],
                                   g1_hbm.at[pl.ds(ka * C, C)], sem_w)
            w2a = pltpu.async_copy(rows2.at[pl.ds(0, C)],
                                   g2_hbm.at[pl.ds(ka * C, C)], sem_w)
            g1b.wait()
            g2b.wait()
            w1b = pltpu.async_copy(rows1.at[pl.ds(C, C)],
                                   g1_hbm.at[pl.ds(kb * C, C)], sem_w)
            w2b = pltpu.async_copy(rows2.at[pl.ds(C, C)],
                                   g2_hbm.at[pl.ds(kb * C, C)], sem_w)
            w1a.wait()
            w2a.wait()
            w1b.wait()
            w2b.wait()

        @pl.when(nj % 2 == 1)
        def _tail():
            kk = start + nj - 1
            pltpu.sync_copy(sd_hbm.at[kk], idx.at[0])
            ga = pltpu.async_copy(tsg_hbm.at[idx.at[0, 0]],
                                  rows1.at[pl.ds(0, C)], sem0)
            gb = pltpu.async_copy(tdx_hbm.at[idx.at[0, 1]],
                                  rows2.at[pl.ds(0, C)], sem1)
            ga.wait()
            gb.wait()
            pltpu.sync_copy(rows1.at[pl.ds(0, C)],
                            g1_hbm.at[pl.ds(kk * C, C)])
            pltpu.sync_copy(rows2.at[pl.ds(0, C)],
                            g2_hbm.at[pl.ds(kk * C, C)])

    return k(T_sg, T_dx, sd)


def _edge_ew(G1, G2, yW2):
    E = G1.shape[0]

    def kern(g1_ref, g2_ref, y_ref, m_ref, sp_ref, s_ref, acc_ref, *, grid):
        i = pl.program_id(0)
        g1 = g1_ref[...]
        m = g1[:, :HID] + g2_ref[:, :HID] + y_ref[...]
        sig = jax.nn.sigmoid(m)
        m_ref[...] = m
        sp_ref[:, :HID] = sig
        sp_ref[:, HID:] = g1[:, HID:] * sig
        _accum_stats(m, i, grid, s_ref, acc_ref)

    m, sp, s = _grid_call(kern, E, [G1, G2, yW2],
                          [2 * HID, 2 * HID, HID], [HID, 2 * HID], 1)
    return m, sp, s


_LG_BS = 4096
_LG_TRASH = 128


def _sc_scatter_lg(SP, sorted_eid, sorted_dst, starts, n):
    E = SP.shape[0]
    C = 128
    BS = _LG_BS
    assert E % C == 0
    nchunks = E // C
    nbins = -(-n // BS)
    nbins = nbins + nbins % SC_NC
    passes = nbins // SC_NC
    stripe = (BS + _LG_TRASH) // SC_NS
    dump = BS // SC_NS
    zeros = jnp.zeros((stripe, 2 * HID), jnp.float32)
    eid2 = sorted_eid.reshape(nchunks, C)
    sdst2 = sorted_dst.reshape(nchunks, C)
    mesh = plsc.VectorSubcoreMesh(core_axis_name="c", subcore_axis_name="s")

    @functools.partial(
        pl.kernel, mesh=mesh,
        out_type=jax.ShapeDtypeStruct((nbins * BS, 2 * HID), jnp.float32),
        scratch_types=[pltpu.VMEM((2, C), jnp.int32),
                       pltpu.VMEM((2, C), jnp.int32),
                       pltpu.VMEM((2, C), jnp.int32),
                       pltpu.VMEM((2 * C, 2 * HID), jnp.float32),
                       pltpu.VMEM_SHARED((BS + _LG_TRASH, 2 * HID),
                                         jnp.float32),
                       pltpu.VMEM((nbins, 16), jnp.int32),
                       pltpu.VMEM((stripe, 2 * HID), jnp.float32),
                       pltpu.SemaphoreType.DMA,
                       pltpu.SemaphoreType.DMA,
                       pltpu.SemaphoreType.DMA],
    )
    def k(sp_hbm, eid_hbm, sdst_hbm, starts_hbm, z_hbm, out_hbm,
          idxd, idxe, locb, rows, acc, st_v, zbuf, sem0, sem1, sem_s):
        c = lax.axis_index("c")
        s = lax.axis_index("s")
        pltpu.sync_copy(starts_hbm, st_v)
        pltpu.sync_copy(z_hbm, zbuf)

        def compute_loc(buf, base_val):
            for i in range(C // 16):
                v = idxd[buf, pl.ds(i * 16, 16)]
                lv = v - base_val
                valid = (lv >= 0) & (lv < BS)
                trash = BS + i * 16 + lax.iota(jnp.int32, 16)
                locb[buf, pl.ds(i * 16, 16)] = jnp.where(valid, lv, trash)

        @pl.loop(0, passes)
        def _pass(p):
            b = p * SC_NC + c
            base_val = b * BS
            pltpu.sync_copy(zbuf, acc.at[pl.ds(s * stripe, stripe)])
            plsc.subcore_barrier()
            row = st_v[b]
            lo = row[0]
            hi = row[1]
            c0 = lo // C
            c1 = (hi + C - 1) // C
            nj = jnp.maximum(0, (c1 - c0 - s + SC_NS - 1) // SC_NS)

            @pl.loop(0, nj // 2)
            def _pair(j2):
                ka = c0 + s + SC_NS * 2 * j2
                kb = ka + SC_NS
                da = pltpu.async_copy(sdst_hbm.at[ka], idxd.at[0], sem0)
                ea = pltpu.async_copy(eid_hbm.at[ka], idxe.at[0], sem0)
                db = pltpu.async_copy(sdst_hbm.at[kb], idxd.at[1], sem1)
                eb = pltpu.async_copy(eid_hbm.at[kb], idxe.at[1], sem1)
                da.wait()
                ea.wait()
                ga = pltpu.async_copy(sp_hbm.at[idxe.at[0]],
                                      rows.at[pl.ds(0, C)], sem0)
                db.wait()
                eb.wait()
                gb = pltpu.async_copy(sp_hbm.at[idxe.at[1]],
                                      rows.at[pl.ds(C, C)], sem1)
                compute_loc(0, base_val)
                ga.wait()
                sa = pltpu.async_copy(rows.at[pl.ds(0, C)],
                                      acc.at[locb.at[0]], sem_s, add=True)
                compute_loc(1, base_val)
                gb.wait()
                sb = pltpu.async_copy(rows.at[pl.ds(C, C)],
                                      acc.at[locb.at[1]], sem_s, add=True)
                sa.wait()
                sb.wait()

            @pl.when(nj % 2 == 1)
            def _tail():
                kk = c0 + s + SC_NS * (nj - 1)
                pltpu.sync_copy(sdst_hbm.at[kk], idxd.at[0])
                pltpu.sync_copy(eid_hbm.at[kk], idxe.at[0])
                cp = pltpu.async_copy(sp_hbm.at[idxe.at[0]],
                                      rows.at[pl.ds(0, C)], sem0)
                compute_loc(0, base_val)
                cp.wait()
                pltpu.sync_copy(rows.at[pl.ds(0, C)], acc.at[locb.at[0]],
                                add=True)

            plsc.subcore_barrier()
            pltpu.sync_copy(
                acc.at[pl.ds(s * dump, dump)],
                out_hbm.at[pl.ds(pl.multiple_of(base_val + s * dump, dump),
                                 dump)])
            plsc.subcore_barrier()

    return k(SP, eid2, sdst2, starts, zeros)


def _egc_layer(x, y, W, b, src, dst, n, lg_sort):
    Wcat = jnp.concatenate([W[0], W[4], W[1], W[3]], axis=1)
    bcat = jnp.concatenate([b[0], b[4], b[1], b[3]])
    T_sg, T_dx = _mm3(x, Wcat, bcat)
    yW2 = _mm(y, W[2], b[2])
    G1, G2 = _gather_rows(T_sg, T_dx, src, dst)
    m, SP, stats_m = _edge_ew(G1, G2, yW2)
    sorted_eid, sorted_dst, starts = lg_sort
    parts = [_sc_scatter_lg(SP, sorted_eid, sorted_dst, starts, n)]
    t, stats_t = _add_div_stats(T_dx, parts)
    x_new = _residual_bnsilu(t, stats_t, x)
    y_new = _residual_bnsilu(m, stats_m, y)
    return x_new, y_new


def kernel(atom_features, r, angle_h, edge_index, lg_edge_index, W_atom,
           b_atom, W_e1, b_e1, W_e2, b_e2, W_a1, b_a1, W_a2, b_a2, egc_W,
           egc_b, W_fc, b_fc):
    src, dst = edge_index[0], edge_index[1]
    lsrc, ldst = lg_edge_index[0], lg_edge_index[1]
    N = atom_features.shape[0]
    E = r.shape[0]

    t, s = _rbf_mm_stats(angle_h.reshape(-1, 1), W_a1, b_a1, -1.0, 1.0, 40,
                         is_r=False)
    t, s = _bnsilu_mm_stats(t, s, W_a2, b_a2)
    z = _bnsilu(t, s)

    t, s = _mm_stats(atom_features, W_atom, b_atom)
    x = _bnsilu(t, s)

    t, s = _rbf_mm_stats(r, W_e1, b_e1, 0.0, 8.0, 80, is_r=True)
    t, s = _bnsilu_mm_stats(t, s, W_e2, b_e2)
    y = _bnsilu(t, s)

    def _sort_meta(d, nseg):
        ne = d.shape[0]
        nbins = -(-nseg // _LG_BS)
        nbins = nbins + nbins % SC_NC
        sd, perm = lax.sort_key_val(d, jnp.arange(ne, dtype=jnp.int32))
        st = jnp.searchsorted(
            sd, jnp.arange(nbins + 1, dtype=jnp.int32) * _LG_BS
        ).astype(jnp.int32)
        st2 = jnp.zeros((nbins, 16), jnp.int32)
        st2 = st2.at[:, 0].set(st[:-1]).at[:, 1].set(st[1:])
        return (perm, sd, st2)

    g_sort = _sort_meta(dst, N)
    lg_sort = _sort_meta(ldst, E)

    x, m = _egc_layer(x, y, egc_W[0], egc_b[0], src, dst, N, g_sort)
    y, z = _egc_layer(m, z, egc_W[1], egc_b[1], lsrc, ldst, E, lg_sort)
    x, m = _egc_layer(x, y, egc_W[2], egc_b[2], src, dst, N, g_sort)
    y, z = _egc_layer(m, z, egc_W[3], egc_b[3], lsrc, ldst, E, lg_sort)
    x, y = _egc_layer(x, y, egc_W[4], egc_b[4], src, dst, N, g_sort)
    x, y = _egc_layer(x, y, egc_W[5], egc_b[5], src, dst, N, g_sort)

    return _readout(x, W_fc, b_fc)
